# Initial kernel scaffold; baseline (speedup 1.0000x reference)
import jax
import jax.numpy as jnp
from jax.experimental import pallas as pl


def kernel(x, edge_index, attn, W_node, b_node, W_ctx, b_ctx, W_upd, b_upd, base_weight, spline_weight, spline_scaler):
    raise NotImplementedError("write your pallas kernel here")


# speedup vs baseline: 3.5883x; 3.5883x over previous
import numpy as np
import jax
import jax.numpy as jnp
from jax import lax
from jax.experimental import pallas as pl
from jax.experimental.pallas import tpu as pltpu
from jax.experimental.pallas import tpu_sc as plsc

_NC, _NS, _L = 2, 16, 16
_NW = _NC * _NS
_CH = 128
_NCH = 80
_EPW = _CH * _NCH
_E_PAD = _NW * _EPW
_RPT = 632
_NROWS = _NS * _RPT
_F = 128
_DW = 8

_GH = 2.0 / 3.0
_T = [float(np.float32(np.float32(i) * np.float32(_GH)) + np.float32(-1.0))
      for i in range(-2, 6)]


def _f32diff(a, b):
    return float(np.float32(np.float32(a) - np.float32(b)))


def _ctx_body(x_ref, w_ref, b_ref, o_ref):
    o_ref[...] = lax.dot_general(
        x_ref[...], w_ref[...], (((1,), (1,)), ((), ())),
        preferred_element_type=jnp.float32) + b_ref[...]


def _ctx_transform(x, w, b):
    n = x.shape[0]
    bm = 1000
    grid = n // bm
    return pl.pallas_call(
        _ctx_body,
        grid=(grid,),
        in_specs=[
            pl.BlockSpec((bm, _F), lambda i: (i, 0)),
            pl.BlockSpec((_F, _F), lambda i: (0, 0)),
            pl.BlockSpec((1, _F), lambda i: (0, 0)),
        ],
        out_specs=pl.BlockSpec((bm, _F), lambda i: (i, 0)),
        out_shape=jax.ShapeDtypeStruct((n, _F), jnp.float32),
    )(x, w, b.reshape(1, _F))


_SPLAT_DNUMS = lax.GatherDimensionNumbers(
    offset_dims=(), collapsed_slice_dims=(0,), start_index_map=(0,))


def _splat(vec, lane):
    idx = jnp.full((_L, 1), lane, dtype=jnp.int32)
    return lax.gather(vec, idx, _SPLAT_DNUMS, (1,),
                      mode=lax.GatherScatterMode.PROMISE_IN_BOUNDS)


def _sc_body(xctx_hbm, src_hbm, dst_hbm, attn_hbm, zeros_hbm,
             agg_hbm, deg_hbm,
             src_v, dst_v, attn_v, rows_v, ones_v, zlin_v, dtmp_v,
             agg_sp, deg_sp, sem):
    c = lax.axis_index("c")
    s = lax.axis_index("s")
    wid = c * _NS + s

    pltpu.sync_copy(src_hbm.at[wid], src_v)
    pltpu.sync_copy(dst_hbm.at[wid], dst_v)
    pltpu.sync_copy(attn_hbm.at[wid], attn_v)

    for i in range(_CH // _L):
        ones_v[pl.ds(i * _L, _L)] = jnp.full((_L,), 1.0, jnp.float32)
        zlin_v[pl.ds(i * _L, _L)] = jnp.zeros((_L,), jnp.float32)
    pltpu.sync_copy(zeros_hbm, rows_v)

    base = s * _RPT
    for off in range(0, _RPT, _CH):
        m = min(_CH, _RPT - off)
        pltpu.sync_copy(rows_v.at[pl.ds(0, m)], agg_sp.at[pl.ds(base + off, m)])
        pltpu.sync_copy(zlin_v.at[pl.ds(0, m)], deg_sp.at[pl.ds(base + off, m)])
    plsc.subcore_barrier()

    def chunk_body(j, carry):
        pltpu.async_copy(xctx_hbm.at[src_v.at[j]], rows_v, sem).wait()
        attn_row = attn_v.at[j]
        for g in range(_CH // _L):
            a16 = attn_row[pl.ds(g * _L, _L)]
            for e in range(_L):
                a = _splat(a16, e)
                rr = rows_v.at[g * _L + e]
                for cb in range(_F // _L):
                    sl = pl.ds(cb * _L, _L)
                    rr[sl] = rr[sl] * a
        pltpu.sync_copy(rows_v, agg_sp.at[dst_v.at[j]], add=True)
        pltpu.sync_copy(ones_v, deg_sp.at[dst_v.at[j]], add=True)
        return carry

    lax.fori_loop(0, _NCH, chunk_body, 0)
    plsc.subcore_barrier()

    pltpu.sync_copy(agg_sp.at[pl.ds(base, _RPT)],
                    agg_hbm.at[c, pl.ds(base, _RPT)])
    pltpu.sync_copy(deg_sp.at[pl.ds(base, _RPT)], dtmp_v)
    pltpu.sync_copy(dtmp_v, deg_hbm.at[pl.ds(c * _NROWS + base, _RPT)])


def _sc_aggregate(x_ctx, src_p, dst_p, attn_p, zeros_hbm):
    mesh = plsc.VectorSubcoreMesh(core_axis_name="c", subcore_axis_name="s")
    return pl.kernel(
        _sc_body,
        out_type=(
            jax.ShapeDtypeStruct((_NC, _NROWS, _F), jnp.float32),
            jax.ShapeDtypeStruct((_NC * _NROWS,), jnp.float32),
        ),
        mesh=mesh,
        scratch_types=[
            pltpu.VMEM((_NCH, _CH), jnp.int32),
            pltpu.VMEM((_NCH, _CH), jnp.int32),
            pltpu.VMEM((_NCH, _CH), jnp.float32),
            pltpu.VMEM((_CH, _F), jnp.float32),
            pltpu.VMEM((_CH,), jnp.float32),
            pltpu.VMEM((_CH,), jnp.float32),
            pltpu.VMEM((_RPT,), jnp.float32),
            pltpu.VMEM_SHARED((_NROWS, _F), jnp.float32),
            pltpu.VMEM_SHARED((_NROWS,), jnp.float32),
            pltpu.SemaphoreType.DMA,
        ],
    )(x_ctx, src_p, dst_p, attn_p, zeros_hbm)


def _post_body(x_ref, agg_ref, deg_ref, wn_ref, bn_ref, wu_ref, bu_ref,
               bw_ref, ws_ref, o_ref):
    x = x_ref[...]
    aggs = agg_ref[0] + agg_ref[1]
    deg = deg_ref[0] + deg_ref[1]
    agg = aggs / jnp.maximum(deg, 1.0)

    x_t = lax.dot_general(x, wn_ref[...], (((1,), (1,)), ((), ())),
                          preferred_element_type=jnp.float32) + bn_ref[...]
    upd = lax.dot_general(x_t + agg, wu_ref[...], (((1,), (1,)), ((), ())),
                          preferred_element_type=jnp.float32) + bu_ref[...]

    silu = upd * (1.0 / (1.0 + jnp.exp(-upd)))
    kan = lax.dot_general(silu, bw_ref[...], (((1,), (1,)), ((), ())),
                          preferred_element_type=jnp.float32)

    b_prev = [((upd >= _T[j]) & (upd < _T[j + 1])).astype(jnp.float32)
              for j in range(7)]
    for k in (1, 2):
        b_cur = []
        for j in range(7 - k):
            dl = _f32diff(_T[j + k], _T[j])
            dr = _f32diff(_T[j + k + 1], _T[j + 1])
            left = (upd - _T[j]) / dl
            right = (_T[j + k + 1] - upd) / dr
            b_cur.append(left * b_prev[j] + right * b_prev[j + 1])
        b_prev = b_cur

    for j in range(5):
        kan = kan + lax.dot_general(
            b_prev[j], ws_ref[j], (((1,), (1,)), ((), ())),
            preferred_element_type=jnp.float32)

    o_ref[...] = jnp.where(deg > 0.0, kan, x)


def _post(x, agg2, deg2, w_node, b_node, w_upd, b_upd, base_w, w_spl):
    n = x.shape[0]
    bm = 1000
    grid = n // bm
    return pl.pallas_call(
        _post_body,
        grid=(grid,),
        in_specs=[
            pl.BlockSpec((bm, _F), lambda i: (i, 0)),
            pl.BlockSpec((2, bm, _F), lambda i: (0, i, 0)),
            pl.BlockSpec((2, bm, 1), lambda i: (0, i, 0)),
            pl.BlockSpec((_F, _F), lambda i: (0, 0)),
            pl.BlockSpec((1, _F), lambda i: (0, 0)),
            pl.BlockSpec((_F, _F), lambda i: (0, 0)),
            pl.BlockSpec((1, _F), lambda i: (0, 0)),
            pl.BlockSpec((_F, _F), lambda i: (0, 0)),
            pl.BlockSpec((5, _F, _F), lambda i: (0, 0, 0)),
        ],
        out_specs=pl.BlockSpec((bm, _F), lambda i: (i, 0)),
        out_shape=jax.ShapeDtypeStruct((n, _F), jnp.float32),
    )(x, agg2, deg2, w_node, b_node, w_upd, b_upd, base_w, w_spl)


def kernel(x, edge_index, attn, W_node, b_node, W_ctx, b_ctx, W_upd, b_upd,
           base_weight, spline_weight, spline_scaler):
    n = x.shape[0]
    e = edge_index.shape[1]
    src = edge_index[0]
    dst = edge_index[1]
    pad = _E_PAD - e
    src_p = jnp.concatenate(
        [src, jnp.zeros((pad,), jnp.int32)]).reshape(_NW, _NCH, _CH)
    dst_p = jnp.concatenate(
        [dst, jnp.full((pad,), n, jnp.int32)]).reshape(_NW, _NCH, _CH)
    attn_p = jnp.concatenate(
        [attn, jnp.zeros((pad,), jnp.float32)]).reshape(_NW, _NCH, _CH)
    zeros_hbm = jnp.zeros((_CH, _F), jnp.float32)

    x_ctx = _ctx_transform(x, W_ctx, b_ctx)
    agg2, deg2 = _sc_aggregate(x_ctx, src_p, dst_p, attn_p, zeros_hbm)
    agg2 = agg2[:, :n, :]
    deg2 = deg2.reshape(2, _NROWS)[:, :n].reshape(2, n, 1)

    w_spl = (spline_weight * spline_scaler[:, :, None]).transpose(2, 0, 1)
    return _post(x, agg2, deg2, W_node, b_node.reshape(1, _F), W_upd,
                 b_upd.reshape(1, _F), base_weight, w_spl)


# speedup vs baseline: 4.4778x; 1.2479x over previous
import numpy as np
import jax
import jax.numpy as jnp
from jax import lax
from jax.experimental import pallas as pl
from jax.experimental.pallas import tpu as pltpu
from jax.experimental.pallas import tpu_sc as plsc

_NC, _NS, _L = 2, 16, 16
_NW = _NC * _NS
_CH = 128
_NCH = 80
_EPW = _CH * _NCH
_E_PAD = _NW * _EPW
_RPT = 632
_NROWS = _NS * _RPT
_F = 128
_DW = 8

_GH = 2.0 / 3.0
_T = [float(np.float32(np.float32(i) * np.float32(_GH)) + np.float32(-1.0))
      for i in range(-2, 6)]


def _f32diff(a, b):
    return float(np.float32(np.float32(a) - np.float32(b)))


def _ctx_body(x_ref, w_ref, b_ref, o_ref):
    o_ref[...] = lax.dot_general(
        x_ref[...], w_ref[...], (((1,), (1,)), ((), ())),
        preferred_element_type=jnp.float32) + b_ref[...]


def _ctx_transform(x, w, b):
    n = x.shape[0]
    bm = 1000
    grid = n // bm
    return pl.pallas_call(
        _ctx_body,
        grid=(grid,),
        in_specs=[
            pl.BlockSpec((bm, _F), lambda i: (i, 0)),
            pl.BlockSpec((_F, _F), lambda i: (0, 0)),
            pl.BlockSpec((1, _F), lambda i: (0, 0)),
        ],
        out_specs=pl.BlockSpec((bm, _F), lambda i: (i, 0)),
        out_shape=jax.ShapeDtypeStruct((n, _F), jnp.float32),
    )(x, w, b.reshape(1, _F))


_SPLAT_DNUMS = lax.GatherDimensionNumbers(
    offset_dims=(), collapsed_slice_dims=(0,), start_index_map=(0,))


def _splat(vec, lane):
    idx = jnp.full((_L, 1), lane, dtype=jnp.int32)
    return lax.gather(vec, idx, _SPLAT_DNUMS, (1,),
                      mode=lax.GatherScatterMode.PROMISE_IN_BOUNDS)


def _sc_body(xctx_hbm, src_hbm, dst_hbm, attn_hbm, zeros_hbm,
             agg_hbm, deg_hbm,
             sidx_v, didx_v, attn_v, rows_v, rows2_v, ones_v, zlin_v, dtmp_v,
             agg_sp, deg_sp, sem, sem2, isem):
    c = lax.axis_index("c")
    s = lax.axis_index("s")
    wid = c * _NS + s
    src_w = src_hbm.at[wid]
    dst_w = dst_hbm.at[wid]
    attn_w = attn_hbm.at[wid]

    for i in range(_CH // _L):
        ones_v[pl.ds(i * _L, _L)] = jnp.full((_L,), 1.0, jnp.float32)
        zlin_v[pl.ds(i * _L, _L)] = jnp.zeros((_L,), jnp.float32)
    pltpu.sync_copy(zeros_hbm, rows_v)

    base = s * _RPT
    for off in range(0, _RPT, _CH):
        m = min(_CH, _RPT - off)
        pltpu.sync_copy(rows_v.at[pl.ds(0, m)], agg_sp.at[pl.ds(base + off, m)])
        pltpu.sync_copy(zlin_v.at[pl.ds(0, m)], deg_sp.at[pl.ds(base + off, m)])
    plsc.subcore_barrier()

    def _process(j, q, rows, gsem):
        @pl.when(j + 2 < _NCH)
        def _():
            qn = q ^ 2
            pltpu.async_copy(src_w.at[j + 2], sidx_v.at[qn], isem)
            pltpu.async_copy(dst_w.at[j + 2], didx_v.at[qn], isem)
            pltpu.async_copy(attn_w.at[j + 2], attn_v.at[qn], isem)

        pltpu.make_async_copy(xctx_hbm.at[sidx_v.at[q]], rows, gsem).wait()
        attn_row = attn_v.at[q]
        for g in range(_CH // _L):
            a16 = attn_row[pl.ds(g * _L, _L)]
            for e in range(_L):
                a = _splat(a16, e)
                rr = rows.at[g * _L + e]
                for cb in range(_F // _L):
                    sl = pl.ds(cb * _L, _L)
                    rr[sl] = rr[sl] * a
        pltpu.sync_copy(rows, agg_sp.at[didx_v.at[q]], add=True)
        pltpu.sync_copy(ones_v, deg_sp.at[didx_v.at[q]], add=True)

        @pl.when(j + 2 < _NCH)
        def _():
            qn = q ^ 2
            pltpu.make_async_copy(src_w.at[j + 2], sidx_v.at[qn], isem).wait()
            pltpu.make_async_copy(dst_w.at[j + 2], didx_v.at[qn], isem).wait()
            pltpu.make_async_copy(attn_w.at[j + 2], attn_v.at[qn], isem).wait()
            pltpu.async_copy(xctx_hbm.at[sidx_v.at[qn]], rows, gsem)

    pltpu.sync_copy(src_w.at[0], sidx_v.at[0])
    pltpu.sync_copy(dst_w.at[0], didx_v.at[0])
    pltpu.sync_copy(attn_w.at[0], attn_v.at[0])
    pltpu.sync_copy(src_w.at[1], sidx_v.at[1])
    pltpu.sync_copy(dst_w.at[1], didx_v.at[1])
    pltpu.sync_copy(attn_w.at[1], attn_v.at[1])
    pltpu.async_copy(xctx_hbm.at[sidx_v.at[0]], rows_v, sem)
    pltpu.async_copy(xctx_hbm.at[sidx_v.at[1]], rows2_v, sem2)

    def chunk_body(jj, carry):
        j = 2 * jj
        q = 2 * (jj & 1)
        _process(j, q, rows_v, sem)
        _process(j + 1, q + 1, rows2_v, sem2)
        return carry

    lax.fori_loop(0, _NCH // 2, chunk_body, 0)
    plsc.subcore_barrier()

    pltpu.sync_copy(agg_sp.at[pl.ds(base, _RPT)],
                    agg_hbm.at[c, pl.ds(base, _RPT)])
    pltpu.sync_copy(deg_sp.at[pl.ds(base, _RPT)], dtmp_v)
    pltpu.sync_copy(dtmp_v, deg_hbm.at[pl.ds(c * _NROWS + base, _RPT)])


def _sc_aggregate(x_ctx, src_p, dst_p, attn_p, zeros_hbm):
    mesh = plsc.VectorSubcoreMesh(core_axis_name="c", subcore_axis_name="s")
    return pl.kernel(
        _sc_body,
        out_type=(
            jax.ShapeDtypeStruct((_NC, _NROWS, _F), jnp.float32),
            jax.ShapeDtypeStruct((_NC * _NROWS,), jnp.float32),
        ),
        mesh=mesh,
        scratch_types=[
            pltpu.VMEM((4, _CH), jnp.int32),
            pltpu.VMEM((4, _CH), jnp.int32),
            pltpu.VMEM((4, _CH), jnp.float32),
            pltpu.VMEM((_CH, _F), jnp.float32),
            pltpu.VMEM((_CH, _F), jnp.float32),
            pltpu.VMEM((_CH,), jnp.float32),
            pltpu.VMEM((_CH,), jnp.float32),
            pltpu.VMEM((_RPT,), jnp.float32),
            pltpu.VMEM_SHARED((_NROWS, _F), jnp.float32),
            pltpu.VMEM_SHARED((_NROWS,), jnp.float32),
            pltpu.SemaphoreType.DMA,
            pltpu.SemaphoreType.DMA,
            pltpu.SemaphoreType.DMA,
        ],
    )(x_ctx, src_p, dst_p, attn_p, zeros_hbm)


def _post_body(x_ref, agg_ref, deg_ref, wn_ref, bn_ref, wu_ref, bu_ref,
               bw_ref, ws_ref, o_ref):
    x = x_ref[...]
    aggs = agg_ref[0] + agg_ref[1]
    deg = deg_ref[0] + deg_ref[1]
    agg = aggs / jnp.maximum(deg, 1.0)

    x_t = lax.dot_general(x, wn_ref[...], (((1,), (1,)), ((), ())),
                          preferred_element_type=jnp.float32) + bn_ref[...]
    upd = lax.dot_general(x_t + agg, wu_ref[...], (((1,), (1,)), ((), ())),
                          preferred_element_type=jnp.float32) + bu_ref[...]

    silu = upd * (1.0 / (1.0 + jnp.exp(-upd)))
    kan = lax.dot_general(silu, bw_ref[...], (((1,), (1,)), ((), ())),
                          preferred_element_type=jnp.float32)

    b_prev = [((upd >= _T[j]) & (upd < _T[j + 1])).astype(jnp.float32)
              for j in range(7)]
    for k in (1, 2):
        b_cur = []
        for j in range(7 - k):
            dl = _f32diff(_T[j + k], _T[j])
            dr = _f32diff(_T[j + k + 1], _T[j + 1])
            left = (upd - _T[j]) / dl
            right = (_T[j + k + 1] - upd) / dr
            b_cur.append(left * b_prev[j] + right * b_prev[j + 1])
        b_prev = b_cur

    for j in range(5):
        kan = kan + lax.dot_general(
            b_prev[j], ws_ref[j], (((1,), (1,)), ((), ())),
            preferred_element_type=jnp.float32)

    o_ref[...] = jnp.where(deg > 0.0, kan, x)


def _post(x, agg2, deg2, w_node, b_node, w_upd, b_upd, base_w, w_spl):
    n = x.shape[0]
    bm = 1000
    grid = n // bm
    return pl.pallas_call(
        _post_body,
        grid=(grid,),
        in_specs=[
            pl.BlockSpec((bm, _F), lambda i: (i, 0)),
            pl.BlockSpec((2, bm, _F), lambda i: (0, i, 0)),
            pl.BlockSpec((2, bm, 1), lambda i: (0, i, 0)),
            pl.BlockSpec((_F, _F), lambda i: (0, 0)),
            pl.BlockSpec((1, _F), lambda i: (0, 0)),
            pl.BlockSpec((_F, _F), lambda i: (0, 0)),
            pl.BlockSpec((1, _F), lambda i: (0, 0)),
            pl.BlockSpec((_F, _F), lambda i: (0, 0)),
            pl.BlockSpec((5, _F, _F), lambda i: (0, 0, 0)),
        ],
        out_specs=pl.BlockSpec((bm, _F), lambda i: (i, 0)),
        out_shape=jax.ShapeDtypeStruct((n, _F), jnp.float32),
    )(x, agg2, deg2, w_node, b_node, w_upd, b_upd, base_w, w_spl)


def kernel(x, edge_index, attn, W_node, b_node, W_ctx, b_ctx, W_upd, b_upd,
           base_weight, spline_weight, spline_scaler):
    n = x.shape[0]
    e = edge_index.shape[1]
    src = edge_index[0]
    dst = edge_index[1]
    pad = _E_PAD - e
    src_p = jnp.concatenate(
        [src, jnp.zeros((pad,), jnp.int32)]).reshape(_NW, _NCH, _CH)
    dst_p = jnp.concatenate(
        [dst, jnp.full((pad,), n, jnp.int32)]).reshape(_NW, _NCH, _CH)
    attn_p = jnp.concatenate(
        [attn, jnp.zeros((pad,), jnp.float32)]).reshape(_NW, _NCH, _CH)
    zeros_hbm = jnp.zeros((_CH, _F), jnp.float32)

    x_ctx = _ctx_transform(x, W_ctx, b_ctx)
    agg2, deg2 = _sc_aggregate(x_ctx, src_p, dst_p, attn_p, zeros_hbm)
    agg2 = agg2[:, :n, :]
    deg2 = deg2.reshape(2, _NROWS)[:, :n].reshape(2, n, 1)

    w_spl = (spline_weight * spline_scaler[:, :, None]).transpose(2, 0, 1)
    return _post(x, agg2, deg2, W_node, b_node.reshape(1, _F), W_upd,
                 b_upd.reshape(1, _F), base_weight, w_spl)
